# baseline (speedup 1.0000x reference)
import functools

import jax
import jax.numpy as jnp
from jax import lax
from jax.experimental import pallas as pl
from jax.experimental.pallas import tpu as pltpu, tpu_sc as plsc

DIM = 64
EPS = 1e-5
NC = 2
NS = 16
LANES = 16
CHUNK = 256
GROUPS = CHUNK // LANES


def _rsqrt(x):
    i = plsc.bitcast(x, jnp.int32)
    i = 0x5F3759DF - (i >> 1)
    y = plsc.bitcast(i, jnp.float32)
    for _ in range(3):
        y = y * (1.5 - 0.5 * x * y * y)
    return y


@functools.cache
def _build(n, vocab):
    n_w = n // (NC * NS)
    n_chunks = n_w // CHUNK
    assert n_w % CHUNK == 0 and n % (NC * NS) == 0 and n_chunks % 2 == 0

    mesh = plsc.VectorSubcoreMesh(
        core_axis_name="c", subcore_axis_name="s",
        num_cores=NC, num_subcores=NS)

    @functools.partial(
        pl.kernel,
        out_type=jax.ShapeDtypeStruct((n, DIM), jnp.float32),
        mesh=mesh,
        scratch_types=[
            pltpu.VMEM((n_w,), jnp.int32),
            pltpu.VMEM((CHUNK, DIM), jnp.float32),
            pltpu.VMEM((CHUNK, DIM), jnp.float32),
            pltpu.VMEM((CHUNK, DIM), jnp.float32),
            pltpu.VMEM((CHUNK, DIM), jnp.float32),
            pltpu.VMEM((DIM * LANES,), jnp.float32),
            pltpu.VMEM((DIM * LANES,), jnp.float32),
            pltpu.SemaphoreType.DMA,
            pltpu.SemaphoreType.DMA,
        ],
        compiler_params=pltpu.CompilerParams(
            needs_layout_passes=False, use_tc_tiling_on_sc=False),
    )
    def emb_ln(idx_hbm, table_hbm, gammat_hbm, betat_hbm, out_hbm,
               idx_v, rows0, rows1, outb0, outb1, gbuf, bbuf, gsem, osem):
        wid = lax.axis_index("s") * NC + lax.axis_index("c")
        base_w = wid * n_w
        pltpu.sync_copy(gammat_hbm, gbuf)
        pltpu.sync_copy(betat_hbm, bbuf)
        pltpu.sync_copy(idx_hbm.at[pl.ds(base_w, n_w)], idx_v)
        lane = lax.iota(jnp.int32, 16)

        def issue_gather(c, rows):
            pltpu.async_copy(
                table_hbm.at[idx_v.at[pl.ds(c * CHUNK, CHUNK)]], rows, gsem)

        def wait_gather(rows):
            pltpu.make_async_copy(table_hbm.at[pl.ds(0, CHUNK)], rows, gsem).wait()

        def out_slice(c):
            return out_hbm.at[pl.ds(base_w + c * CHUNK, CHUNK)]

        def issue_out(c, outb):
            pltpu.async_copy(outb, out_slice(c), osem)

        def wait_out(outb):
            pltpu.make_async_copy(outb, out_slice(0), osem).wait()

        def compute(rows, outb):
            def group_body(g, gcarry):
                rvec = lane + g * LANES
                s = jnp.zeros((16,), jnp.float32)
                q = jnp.zeros((16,), jnp.float32)
                for j in range(DIM):
                    cj = jnp.full((16,), j, jnp.int32)
                    x = plsc.load_gather(rows, [rvec, cj])
                    s = s + x
                    q = q + x * x
                mean = s * (1.0 / DIM)
                var = q * (1.0 / DIM) - mean * mean
                inv = _rsqrt(var + EPS)
                m2 = mean * inv
                for j in range(DIM):
                    cj = jnp.full((16,), j, jnp.int32)
                    x = plsc.load_gather(rows, [rvec, cj])
                    z = x * inv - m2
                    z = z * gbuf[pl.ds(j * LANES, LANES)] + bbuf[pl.ds(j * LANES, LANES)]
                    plsc.store_scatter(outb, [rvec, cj], z)
                return gcarry

            lax.fori_loop(0, GROUPS, group_body, 0)

        issue_gather(0, rows0)

        def body2(c2, carry):
            for p, rows, outb, nrows in ((0, rows0, outb0, rows1),
                                         (1, rows1, outb1, rows0)):
                c = c2 * 2 + p
                wait_gather(rows)

                @pl.when(c + 1 < n_chunks)
                def _():
                    issue_gather(c + 1, nrows)

                @pl.when(c >= 2)
                def _():
                    wait_out(outb)

                compute(rows, outb)
                issue_out(c, outb)
            return carry

        lax.fori_loop(0, n_chunks // 2, body2, 0)
        wait_out(outb0)
        wait_out(outb1)

    return emb_ln


def kernel(tcword_id, table, gamma, beta):
    b, l = tcword_id.shape
    idx = tcword_id.reshape(-1).astype(jnp.int32)
    gammat = jnp.broadcast_to(gamma[:, None], (DIM, LANES)).reshape(-1)
    betat = jnp.broadcast_to(beta[:, None], (DIM, LANES)).reshape(-1)
    fn = _build(b * l, table.shape[0])
    out = fn(idx, table, gammat, betat)
    return out.reshape(b, l, DIM)


# speedup vs baseline: 1.3998x; 1.3998x over previous
import functools

import jax
import jax.numpy as jnp
from jax import lax
from jax.experimental import pallas as pl
from jax.experimental.pallas import tpu as pltpu, tpu_sc as plsc

DIM = 64
EPS = 1e-5
NC = 2
NS = 16
LANES = 16
CHUNK = 256
GROUPS = CHUNK // LANES


def _rsqrt(x):
    i = plsc.bitcast(x, jnp.int32)
    i = 0x5F3759DF - (i >> 1)
    y = plsc.bitcast(i, jnp.float32)
    for _ in range(3):
        y = y * (1.5 - 0.5 * x * y * y)
    return y


@functools.cache
def _build(n, vocab):
    n_w = n // (NC * NS)
    n_chunks = n_w // CHUNK
    assert n_w % CHUNK == 0 and n % (NC * NS) == 0 and n_chunks % 2 == 0

    mesh = plsc.VectorSubcoreMesh(
        core_axis_name="c", subcore_axis_name="s",
        num_cores=NC, num_subcores=NS)

    @functools.partial(
        pl.kernel,
        out_type=jax.ShapeDtypeStruct((n, DIM), jnp.float32),
        mesh=mesh,
        scratch_types=[
            pltpu.VMEM((n_w,), jnp.int32),
            pltpu.VMEM((CHUNK, DIM), jnp.float32),
            pltpu.VMEM((CHUNK, DIM), jnp.float32),
            pltpu.VMEM((CHUNK, DIM), jnp.float32),
            pltpu.VMEM((CHUNK, DIM), jnp.float32),
            pltpu.VMEM((DIM * LANES,), jnp.float32),
            pltpu.VMEM((DIM * LANES,), jnp.float32),
            pltpu.SemaphoreType.DMA,
            pltpu.SemaphoreType.DMA,
        ],
        compiler_params=pltpu.CompilerParams(
            needs_layout_passes=False, use_tc_tiling_on_sc=False),
    )
    def emb_ln(idx_hbm, table_hbm, gammat_hbm, betat_hbm, out_hbm,
               idx_v, rows0, rows1, outb0, outb1, gbuf, bbuf, gsem, osem):
        wid = lax.axis_index("s") * NC + lax.axis_index("c")
        base_w = wid * n_w
        pltpu.sync_copy(gammat_hbm, gbuf)
        pltpu.sync_copy(betat_hbm, bbuf)
        pltpu.sync_copy(idx_hbm.at[pl.ds(base_w, n_w)], idx_v)
        lane = lax.iota(jnp.int32, 16)

        def issue_gather(c, rows):
            pltpu.async_copy(
                table_hbm.at[idx_v.at[pl.ds(c * CHUNK, CHUNK)]], rows, gsem)

        def wait_gather(rows):
            pltpu.make_async_copy(table_hbm.at[pl.ds(0, CHUNK)], rows, gsem).wait()

        def out_slice(c):
            return out_hbm.at[pl.ds(base_w + c * CHUNK, CHUNK)]

        def issue_out(c, outb):
            pltpu.async_copy(outb, out_slice(c), osem)

        def wait_out(outb):
            pltpu.make_async_copy(outb, out_slice(0), osem).wait()

        def compute(rows, outb):
            @plsc.parallel_loop(0, GROUPS)
            def group_body(g):
                rvec = lane + g * LANES
                acc = [jnp.zeros((16,), jnp.float32) for _ in range(8)]
                for j in range(DIM):
                    cj = jnp.full((16,), j, jnp.int32)
                    x = plsc.load_gather(rows, [rvec, cj])
                    acc[j % 4] = acc[j % 4] + x
                    acc[4 + j % 4] = acc[4 + j % 4] + x * x
                s = (acc[0] + acc[1]) + (acc[2] + acc[3])
                q = (acc[4] + acc[5]) + (acc[6] + acc[7])
                mean = s * (1.0 / DIM)
                var = q * (1.0 / DIM) - mean * mean
                inv = _rsqrt(var + EPS)
                m2 = mean * inv

                @plsc.parallel_loop(0, DIM, unroll=8)
                def col_body(j):
                    cj = jnp.full((16,), j, jnp.int32)
                    x = plsc.load_gather(rows, [rvec, cj])
                    z = x * inv - m2
                    z = (z * gbuf[pl.ds(j * LANES, LANES)]
                         + bbuf[pl.ds(j * LANES, LANES)])
                    plsc.store_scatter(outb, [rvec, cj], z)

        issue_gather(0, rows0)

        def body2(c2, carry):
            for p, rows, outb, nrows in ((0, rows0, outb0, rows1),
                                         (1, rows1, outb1, rows0)):
                c = c2 * 2 + p
                wait_gather(rows)

                @pl.when(c + 1 < n_chunks)
                def _():
                    issue_gather(c + 1, nrows)

                @pl.when(c >= 2)
                def _():
                    wait_out(outb)

                compute(rows, outb)
                issue_out(c, outb)
            return carry

        lax.fori_loop(0, n_chunks // 2, body2, 0)
        wait_out(outb0)
        wait_out(outb1)

    return emb_ln


def kernel(tcword_id, table, gamma, beta):
    b, l = tcword_id.shape
    idx = tcword_id.reshape(-1).astype(jnp.int32)
    gammat = jnp.broadcast_to(gamma[:, None], (DIM, LANES)).reshape(-1)
    betat = jnp.broadcast_to(beta[:, None], (DIM, LANES)).reshape(-1)
    fn = _build(b * l, table.shape[0])
    out = fn(idx, table, gammat, betat)
    return out.reshape(b, l, DIM)


# speedup vs baseline: 2.7712x; 1.9798x over previous
import functools

import jax
import jax.numpy as jnp
from jax import lax
from jax.experimental import pallas as pl
from jax.experimental.pallas import tpu as pltpu, tpu_sc as plsc

DIM = 64
EPS = 1e-5
NC = 2
NS = 16
LANES = 16
CHUNK = 256
GROUPS = CHUNK // LANES


def _rsqrt(x):
    i = plsc.bitcast(x, jnp.int32)
    i = 0x5F3759DF - (i >> 1)
    y = plsc.bitcast(i, jnp.float32)
    for _ in range(3):
        y = y * (1.5 - 0.5 * x * y * y)
    return y


@functools.cache
def _build(n, vocab):
    n_w = n // (NC * NS)
    n_chunks = n_w // CHUNK
    assert n_w % CHUNK == 0 and n % (NC * NS) == 0 and n_chunks % 2 == 0

    mesh = plsc.VectorSubcoreMesh(
        core_axis_name="c", subcore_axis_name="s",
        num_cores=NC, num_subcores=NS)

    @functools.partial(
        pl.kernel,
        out_type=jax.ShapeDtypeStruct((n, DIM), jnp.float32),
        mesh=mesh,
        scratch_types=[
            pltpu.VMEM((n_w,), jnp.int32),
            pltpu.VMEM((CHUNK, DIM), jnp.float32),
            pltpu.VMEM((CHUNK, DIM), jnp.float32),
            pltpu.VMEM((CHUNK, DIM), jnp.float32),
            pltpu.VMEM((CHUNK, DIM), jnp.float32),
            pltpu.VMEM((DIM,), jnp.float32),
            pltpu.VMEM((DIM,), jnp.float32),
            pltpu.SemaphoreType.DMA,
            pltpu.SemaphoreType.DMA,
        ],
        compiler_params=pltpu.CompilerParams(
            needs_layout_passes=False, use_tc_tiling_on_sc=False),
    )
    def emb_ln(idx_hbm, table_hbm, gammat_hbm, betat_hbm, out_hbm,
               idx_v, rows0, rows1, outb0, outb1, gbuf, bbuf, gsem, osem):
        wid = lax.axis_index("s") * NC + lax.axis_index("c")
        base_w = wid * n_w
        pltpu.sync_copy(gammat_hbm, gbuf)
        pltpu.sync_copy(betat_hbm, bbuf)
        pltpu.sync_copy(idx_hbm.at[pl.ds(base_w, n_w)], idx_v)
        lane = lax.iota(jnp.int32, 16)

        def issue_gather(c, rows):
            pltpu.async_copy(
                table_hbm.at[idx_v.at[pl.ds(c * CHUNK, CHUNK)]], rows, gsem)

        def wait_gather(rows):
            pltpu.make_async_copy(table_hbm.at[pl.ds(0, CHUNK)], rows, gsem).wait()

        def out_slice(c):
            return out_hbm.at[pl.ds(base_w + c * CHUNK, CHUNK)]

        def issue_out(c, outb):
            pltpu.async_copy(outb, out_slice(c), osem)

        def wait_out(outb):
            pltpu.make_async_copy(outb, out_slice(0), osem).wait()

        def compute(rows, outb):
            @plsc.parallel_loop(0, GROUPS)
            def group_body(g):
                rvec = lane + g * LANES
                acc = [jnp.zeros((16,), jnp.float32) for _ in range(8)]
                for j in range(DIM):
                    cj = (lane + j) & (DIM - 1)
                    x = plsc.load_gather(rows, [rvec, cj])
                    acc[j % 4] = acc[j % 4] + x
                    acc[4 + j % 4] = acc[4 + j % 4] + x * x
                s = (acc[0] + acc[1]) + (acc[2] + acc[3])
                q = (acc[4] + acc[5]) + (acc[6] + acc[7])
                mean = s * (1.0 / DIM)
                var = q * (1.0 / DIM) - mean * mean
                inv = _rsqrt(var + EPS)
                m2 = mean * inv

                @plsc.parallel_loop(0, DIM, unroll=8)
                def col_body(j):
                    cj = (lane + j) & (DIM - 1)
                    x = plsc.load_gather(rows, [rvec, cj])
                    z = x * inv - m2
                    z = (z * plsc.load_gather(gbuf, [cj])
                         + plsc.load_gather(bbuf, [cj]))
                    plsc.store_scatter(outb, [rvec, cj], z)

        issue_gather(0, rows0)

        def body2(c2, carry):
            for p, rows, outb, nrows in ((0, rows0, outb0, rows1),
                                         (1, rows1, outb1, rows0)):
                c = c2 * 2 + p
                wait_gather(rows)

                @pl.when(c + 1 < n_chunks)
                def _():
                    issue_gather(c + 1, nrows)

                @pl.when(c >= 2)
                def _():
                    wait_out(outb)

                compute(rows, outb)
                issue_out(c, outb)
            return carry

        lax.fori_loop(0, n_chunks // 2, body2, 0)
        wait_out(outb0)
        wait_out(outb1)

    return emb_ln


def kernel(tcword_id, table, gamma, beta):
    b, l = tcword_id.shape
    idx = tcword_id.reshape(-1).astype(jnp.int32)
    fn = _build(b * l, table.shape[0])
    out = fn(idx, table, gamma, beta)
    return out.reshape(b, l, DIM)


# speedup vs baseline: 4.0842x; 1.4738x over previous
import functools

import jax
import jax.numpy as jnp
from jax import lax
from jax.experimental import pallas as pl
from jax.experimental.pallas import tpu as pltpu, tpu_sc as plsc

B = 16384
L = 50
DIM = 64
EPS = 1e-5
NC = 2
NS = 16
NW = NC * NS
LANES = 16
BBLK = 128
BPW = B // BBLK // NW
LP = L // 2
CHUNK = 2 * BBLK
GROUPS = CHUNK // LANES


def _rsqrt(x):
    i = plsc.bitcast(x, jnp.int32)
    i = 0x5F3759DF - (i >> 1)
    y = plsc.bitcast(i, jnp.float32)
    for _ in range(3):
        y = y * (1.5 - 0.5 * x * y * y)
    return y


@functools.cache
def _build(vocab):
    n = B * L
    n_w = n // NW
    n_chunks = n_w // CHUNK

    mesh = plsc.VectorSubcoreMesh(
        core_axis_name="c", subcore_axis_name="s",
        num_cores=NC, num_subcores=NS)

    @functools.partial(
        pl.kernel,
        out_type=jax.ShapeDtypeStruct((L, DIM // 8, B // BBLK, 8 * BBLK),
                                      jnp.float32),
        mesh=mesh,
        scratch_types=[
            pltpu.VMEM((n_w,), jnp.int32),
            pltpu.VMEM((n_w,), jnp.int32),
            pltpu.VMEM((CHUNK, DIM), jnp.float32),
            pltpu.VMEM((CHUNK, DIM), jnp.float32),
            pltpu.VMEM((16, 8 * BBLK), jnp.float32),
            pltpu.VMEM((16, 8 * BBLK), jnp.float32),
            pltpu.VMEM((DIM,), jnp.float32),
            pltpu.VMEM((DIM,), jnp.float32),
            pltpu.SemaphoreType.DMA,
            pltpu.SemaphoreType.DMA,
        ],
        compiler_params=pltpu.CompilerParams(
            needs_layout_passes=False, use_tc_tiling_on_sc=False),
    )
    def emb_ln(idx_hbm, table_hbm, gamma_hbm, beta_hbm, out_hbm,
               idx_v, idx_r, rows0, rows1, outb0, outb1, gbuf, bbuf,
               gsem, osem):
        wid = lax.axis_index("s") * NC + lax.axis_index("c")
        base_w = wid * n_w
        pltpu.sync_copy(gamma_hbm, gbuf)
        pltpu.sync_copy(beta_hbm, bbuf)
        pltpu.sync_copy(idx_hbm.at[pl.ds(base_w, n_w)], idx_v)
        lane = lax.iota(jnp.int32, 16)

        @plsc.parallel_loop(0, n_w // LANES, unroll=4)
        def reorder(t):
            bq = t % (BBLK // LANES)
            r = t // (BBLK // LANES)
            ll = r % 2
            r = r // 2
            lp = r % LP
            blk = r // LP
            bvec = bq * LANES + lane
            src = blk * (L * BBLK) + bvec * L + lp * 2 + ll
            idx_r[pl.ds(t * LANES, LANES)] = plsc.load_gather(idx_v, [src])

        def issue_gather(c, rows):
            pltpu.async_copy(
                table_hbm.at[idx_r.at[pl.ds(c * CHUNK, CHUNK)]], rows, gsem)

        def wait_gather(rows):
            pltpu.make_async_copy(
                table_hbm.at[pl.ds(0, CHUNK)], rows, gsem).wait()

        def issue_out(c, outb):
            blk = c // LP
            lp = c % LP
            bh = wid * BPW + blk
            for i in range(16):
                pltpu.async_copy(
                    outb.at[i],
                    out_hbm.at[lp * 2 + i // 8, i % 8, bh], osem)

        def wait_out(outb):
            pltpu.make_async_copy(
                out_hbm.at[0, 0, pl.ds(0, 16)], outb, osem).wait()

        def compute(rows, outb):
            @plsc.parallel_loop(0, GROUPS)
            def group_body(g):
                rvec = lane + g * LANES
                acc = [jnp.zeros((16,), jnp.float32) for _ in range(8)]
                for j in range(DIM):
                    cj = (lane + j) & (DIM - 1)
                    x = plsc.load_gather(rows, [rvec, cj])
                    acc[j % 4] = acc[j % 4] + x
                    acc[4 + j % 4] = acc[4 + j % 4] + x * x
                s = (acc[0] + acc[1]) + (acc[2] + acc[3])
                q = (acc[4] + acc[5]) + (acc[6] + acc[7])
                mean = s * (1.0 / DIM)
                var = q * (1.0 / DIM) - mean * mean
                inv = _rsqrt(var + EPS)
                m2 = mean * inv
                obase = (g // 8) * (64 * BBLK) + (g % 8) * LANES

                @plsc.parallel_loop(0, DIM, unroll=8)
                def col_body(j):
                    cj = (lane + j) & (DIM - 1)
                    x = plsc.load_gather(rows, [rvec, cj])
                    z = x * inv - m2
                    z = (z * plsc.load_gather(gbuf, [cj])
                         + plsc.load_gather(bbuf, [cj]))
                    off = obase + cj * BBLK + lane
                    plsc.store_scatter(
                        outb, [off >> 10, off & 1023], z)

        issue_gather(0, rows0)

        def body2(c2, carry):
            for p, rows, outb, nrows in ((0, rows0, outb0, rows1),
                                         (1, rows1, outb1, rows0)):
                c = c2 * 2 + p
                wait_gather(rows)

                @pl.when(c + 1 < n_chunks)
                def _():
                    issue_gather(c + 1, nrows)

                @pl.when(c >= 2)
                def _():
                    wait_out(outb)

                compute(rows, outb)
                issue_out(c, outb)
            return carry

        lax.fori_loop(0, n_chunks // 2, body2, 0)
        wait_out(outb0)
        wait_out(outb1)

    return emb_ln


def kernel(tcword_id, table, gamma, beta):
    b, l = tcword_id.shape
    assert (b, l) == (B, L) and table.shape[1] == DIM
    idx = tcword_id.reshape(-1).astype(jnp.int32)
    fn = _build(table.shape[0])
    out = fn(idx, table, gamma, beta)
    out = out.reshape(L, DIM // 8, B // BBLK, 8, BBLK)
    return out.transpose((2, 4, 0, 1, 3)).reshape(B, L, DIM)
